# baseline (device time: 12430 ns/iter reference)
import jax
import jax.numpy as jnp
from jax import lax
from jax.experimental import pallas as pl
from jax.experimental.pallas import tpu as pltpu


def kernel(x, W, labels):
    T, D = x.shape
    V = W.shape[1]

    def body(x_hbm, w_hbm, lab_hbm, out_hbm,
             xv, labv, wv, nll_v, send_ref, recv_ref,
             copy_sems, out_sem, send_sem, recv_sem):
        my_x = lax.axis_index("x")
        my_y = lax.axis_index("y")
        my_z = lax.axis_index("z")
        partner = (1 - my_x, my_y, my_z)
        barrier = pltpu.get_barrier_semaphore()
        pl.semaphore_signal(barrier, inc=1, device_id=partner,
                            device_id_type=pl.DeviceIdType.MESH)

        cp_x = pltpu.make_async_copy(x_hbm, xv, copy_sems.at[0])
        cp_x.start()
        cp_l = pltpu.make_async_copy(lab_hbm, labv, copy_sems.at[1])
        cp_l.start()
        cp_w = pltpu.make_async_copy(w_hbm, wv, copy_sems.at[2])
        cp_w.start()
        cp_w.wait()
        cp_x.wait()
        cp_l.wait()

        logitsT = lax.dot_general(
            wv[:, :], xv[:, :],
            dimension_numbers=(((0,), (1,)), ((), ())),
            preferred_element_type=jnp.float32)
        s = jnp.sum(jnp.exp(logitsT), axis=0, keepdims=True)
        row = lax.broadcasted_iota(jnp.int32, logitsT.shape, 0)
        lab_local = labv[:, :] - my_x * V
        t = jnp.sum(jnp.where(row == lab_local, logitsT, 0.0),
                    axis=0, keepdims=True)

        send_ref[:, 0:T] = s
        send_ref[:, T:2 * T] = t
        pl.semaphore_wait(barrier, 1)
        rdma = pltpu.make_async_remote_copy(
            src_ref=send_ref, dst_ref=recv_ref,
            send_sem=send_sem, recv_sem=recv_sem,
            device_id=partner, device_id_type=pl.DeviceIdType.MESH)
        rdma.start()
        rdma.wait_recv()
        s_p = recv_ref[:, 0:T]
        t_p = recv_ref[:, T:2 * T]
        nll = jnp.log(s + s_p) - (t + t_p)
        nll_v[:] = nll[0, :]
        cp_out = pltpu.make_async_copy(nll_v, out_hbm, out_sem)
        cp_out.start()
        rdma.wait_send()
        cp_out.wait()

    out = pl.pallas_call(
        body,
        out_shape=jax.ShapeDtypeStruct((T,), jnp.float32),
        in_specs=[
            pl.BlockSpec(memory_space=pl.ANY),
            pl.BlockSpec(memory_space=pl.ANY),
            pl.BlockSpec(memory_space=pl.ANY),
        ],
        out_specs=pl.BlockSpec(memory_space=pl.ANY),
        scratch_shapes=[
            pltpu.VMEM((T, D), jnp.float32),
            pltpu.VMEM((1, T), jnp.int32),
            pltpu.VMEM((D, V), jnp.float32),
            pltpu.VMEM((T,), jnp.float32),
            pltpu.VMEM((1, 2 * T), jnp.float32),
            pltpu.VMEM((1, 2 * T), jnp.float32),
            pltpu.SemaphoreType.DMA((3,)),
            pltpu.SemaphoreType.DMA,
            pltpu.SemaphoreType.DMA,
            pltpu.SemaphoreType.DMA,
        ],
        compiler_params=pltpu.CompilerParams(collective_id=0),
    )(x, W, labels.reshape(1, T))
    return out


# device time: 12290 ns/iter; 1.0114x vs baseline; 1.0114x over previous
import jax
import jax.numpy as jnp
from jax import lax
from jax.experimental import pallas as pl
from jax.experimental.pallas import tpu as pltpu


def kernel(x, W, labels):
    T, D = x.shape
    V = W.shape[1]

    def body(x_hbm, w_hbm, lab_hbm, out_hbm,
             xv, labv, wv, nll_v, send_ref, recv_ref,
             copy_sems, out_sem, send_sem, recv_sem):
        my_x = lax.axis_index("x")
        my_y = lax.axis_index("y")
        my_z = lax.axis_index("z")
        partner = (1 - my_x, my_y, my_z)
        barrier = pltpu.get_barrier_semaphore()
        pl.semaphore_signal(barrier, inc=1, device_id=partner,
                            device_id_type=pl.DeviceIdType.MESH)

        cp_x = pltpu.make_async_copy(x_hbm, xv, copy_sems.at[0])
        cp_x.start()
        cp_l = pltpu.make_async_copy(lab_hbm, labv, copy_sems.at[1])
        cp_l.start()
        cp_w = pltpu.make_async_copy(w_hbm, wv, copy_sems.at[2])
        cp_w.start()
        cp_w.wait()
        cp_x.wait()
        cp_l.wait()

        NSPLIT = 4
        CV = V // NSPLIT
        s = jnp.zeros((1, T), jnp.float32)
        t = jnp.zeros((1, T), jnp.float32)
        for h in range(NSPLIT):
            logitsT = lax.dot_general(
                wv[:, h * CV:(h + 1) * CV], xv[:, :],
                dimension_numbers=(((0,), (1,)), ((), ())),
                preferred_element_type=jnp.float32)
            s = s + jnp.sum(jnp.exp(logitsT), axis=0, keepdims=True)
            row = lax.broadcasted_iota(jnp.int32, logitsT.shape, 0)
            lab_local = labv[:, :] - (my_x * V + h * CV)
            t = t + jnp.sum(jnp.where(row == lab_local, logitsT, 0.0),
                            axis=0, keepdims=True)

        send_ref[:, 0:T] = s
        send_ref[:, T:2 * T] = t
        pl.semaphore_wait(barrier, 1)
        rdma = pltpu.make_async_remote_copy(
            src_ref=send_ref, dst_ref=recv_ref,
            send_sem=send_sem, recv_sem=recv_sem,
            device_id=partner, device_id_type=pl.DeviceIdType.MESH)
        rdma.start()
        rdma.wait_recv()
        s_p = recv_ref[:, 0:T]
        t_p = recv_ref[:, T:2 * T]
        nll = jnp.log(s + s_p) - (t + t_p)
        nll_v[:] = nll[0, :]
        cp_out = pltpu.make_async_copy(nll_v, out_hbm, out_sem)
        cp_out.start()
        rdma.wait_send()
        cp_out.wait()

    out = pl.pallas_call(
        body,
        out_shape=jax.ShapeDtypeStruct((T,), jnp.float32),
        in_specs=[
            pl.BlockSpec(memory_space=pl.ANY),
            pl.BlockSpec(memory_space=pl.ANY),
            pl.BlockSpec(memory_space=pl.ANY),
        ],
        out_specs=pl.BlockSpec(memory_space=pl.ANY),
        scratch_shapes=[
            pltpu.VMEM((T, D), jnp.float32),
            pltpu.VMEM((1, T), jnp.int32),
            pltpu.VMEM((D, V), jnp.float32),
            pltpu.VMEM((T,), jnp.float32),
            pltpu.VMEM((1, 2 * T), jnp.float32),
            pltpu.VMEM((1, 2 * T), jnp.float32),
            pltpu.SemaphoreType.DMA((3,)),
            pltpu.SemaphoreType.DMA,
            pltpu.SemaphoreType.DMA,
            pltpu.SemaphoreType.DMA,
        ],
        compiler_params=pltpu.CompilerParams(collective_id=0),
    )(x, W, labels.reshape(1, T))
    return out


# device time: 12108 ns/iter; 1.0266x vs baseline; 1.0150x over previous
import jax
import jax.numpy as jnp
from jax import lax
from jax.experimental import pallas as pl
from jax.experimental.pallas import tpu as pltpu


def kernel(x, W, labels):
    T, D = x.shape
    V = W.shape[1]

    def body(x_hbm, w_hbm, lab_hbm, out_hbm,
             xv, labv, wv, nll_v, send_ref, recv_ref,
             copy_sems, out_sem, send_sem, recv_sem):
        my_x = lax.axis_index("x")
        my_y = lax.axis_index("y")
        my_z = lax.axis_index("z")
        partner = (1 - my_x, my_y, my_z)
        barrier = pltpu.get_barrier_semaphore()
        pl.semaphore_signal(barrier, inc=1, device_id=partner,
                            device_id_type=pl.DeviceIdType.MESH)

        cp_x = pltpu.make_async_copy(x_hbm, xv, copy_sems.at[0])
        cp_x.start()
        cp_l = pltpu.make_async_copy(lab_hbm, labv, copy_sems.at[1])
        cp_l.start()
        cp_w = pltpu.make_async_copy(w_hbm, wv, copy_sems.at[2])
        cp_w.start()
        cp_w.wait()
        cp_x.wait()
        cp_l.wait()

        NSPLIT = 8
        CV = V // NSPLIT
        s = jnp.zeros((1, T), jnp.float32)
        t = jnp.zeros((1, T), jnp.float32)
        for h in range(NSPLIT):
            logitsT = lax.dot_general(
                wv[:, h * CV:(h + 1) * CV], xv[:, :],
                dimension_numbers=(((0,), (1,)), ((), ())),
                preferred_element_type=jnp.float32)
            s = s + jnp.sum(jnp.exp(logitsT), axis=0, keepdims=True)
            row = lax.broadcasted_iota(jnp.int32, logitsT.shape, 0)
            lab_local = labv[:, :] - (my_x * V + h * CV)
            t = t + jnp.sum(jnp.where(row == lab_local, logitsT, 0.0),
                            axis=0, keepdims=True)

        send_ref[:, 0:T] = s
        send_ref[:, T:2 * T] = t
        pl.semaphore_wait(barrier, 1)
        rdma = pltpu.make_async_remote_copy(
            src_ref=send_ref, dst_ref=recv_ref,
            send_sem=send_sem, recv_sem=recv_sem,
            device_id=partner, device_id_type=pl.DeviceIdType.MESH)
        rdma.start()
        rdma.wait_recv()
        s_p = recv_ref[:, 0:T]
        t_p = recv_ref[:, T:2 * T]
        nll = jnp.log(s + s_p) - (t + t_p)
        nll_v[:] = nll[0, :]
        cp_out = pltpu.make_async_copy(nll_v, out_hbm, out_sem)
        cp_out.start()
        rdma.wait_send()
        cp_out.wait()

    out = pl.pallas_call(
        body,
        out_shape=jax.ShapeDtypeStruct((T,), jnp.float32),
        in_specs=[
            pl.BlockSpec(memory_space=pl.ANY),
            pl.BlockSpec(memory_space=pl.ANY),
            pl.BlockSpec(memory_space=pl.ANY),
        ],
        out_specs=pl.BlockSpec(memory_space=pl.ANY),
        scratch_shapes=[
            pltpu.VMEM((T, D), jnp.float32),
            pltpu.VMEM((1, T), jnp.int32),
            pltpu.VMEM((D, V), jnp.float32),
            pltpu.VMEM((T,), jnp.float32),
            pltpu.VMEM((1, 2 * T), jnp.float32),
            pltpu.VMEM((1, 2 * T), jnp.float32),
            pltpu.SemaphoreType.DMA((3,)),
            pltpu.SemaphoreType.DMA,
            pltpu.SemaphoreType.DMA,
            pltpu.SemaphoreType.DMA,
        ],
        compiler_params=pltpu.CompilerParams(collective_id=0),
    )(x, W, labels.reshape(1, T))
    return out
